# baseline (speedup 1.0000x reference)
import functools

import jax
import jax.numpy as jnp
from jax import lax
from jax.experimental import pallas as pl
from jax.experimental.pallas import tpu as pltpu
from jax.experimental.pallas import tpu_sc as plsc

VOCAB = 100000
D = 128
B = 1024
L = 200
EPS = 1e-12

NC = 2
NS = 16
NW = NC * NS
N_TOK = B * L
TOK_PER_W = N_TOK // NW
CH = 128
N_CHUNKS = TOK_PER_W // CH
NBUF = 2

@functools.cache
def _make_sc_gather():
    mesh = plsc.VectorSubcoreMesh(
        core_axis_name="c", subcore_axis_name="s", num_cores=NC, num_subcores=NS
    )
    return functools.partial(
        pl.kernel,
        out_type=jax.ShapeDtypeStruct((N_TOK, D), jnp.float32),
        mesh=mesh,
        scratch_types=[
            pltpu.VMEM((N_CHUNKS, CH), jnp.int32),
            pltpu.VMEM((NBUF, CH, D), jnp.float32),
            pltpu.SemaphoreType.DMA,
            pltpu.SemaphoreType.DMA,
        ],
    )(_sc_gather_body)


def _sc_gather_body(ids_hbm, table_hbm, out_hbm, idx_v, rows_v, sem0, sem1):
    wid = lax.axis_index("s") * NC + lax.axis_index("c")
    out_base = wid * TOK_PER_W
    sems = (sem0, sem1)
    pltpu.sync_copy(ids_hbm.at[wid], idx_v)

    def start(chunk, buf):
        return pltpu.async_copy(
            table_hbm.at[idx_v.at[chunk]], rows_v.at[buf], sems[buf]
        )

    for b in range(NBUF):
        start(b, b)

    def body(c, carry):
        for b in range(NBUF):
            chunk = c + b
            pltpu.make_async_copy(
                table_hbm.at[idx_v.at[chunk]], rows_v.at[b], sems[b]
            ).wait()
            pltpu.sync_copy(
                rows_v.at[b], out_hbm.at[pl.ds(out_base + chunk * CH, CH)]
            )

            @pl.when(chunk + NBUF < N_CHUNKS)
            def _():
                start(chunk + NBUF, b)

        return carry

    lax.fori_loop(0, N_CHUNKS // NBUF, lambda i, cy: body(i * NBUF, cy), 0,
                  unroll=False)


def _tc_posln_body(x_ref, w_ref, b_ref, o_ref, pos_scr):
    @pl.when(pl.program_id(0) == 0)
    def _():
        l_idx = lax.broadcasted_iota(jnp.int32, (L, D // 2), 0).astype(jnp.float32)
        j_idx = lax.broadcasted_iota(jnp.int32, (L, D // 2), 1).astype(jnp.float32)
        inv_freq = jnp.exp(j_idx * (-2.0 / D * jnp.log(10000.0)))
        angle = (L - 1.0 - l_idx) * inv_freq
        pos_scr[...] = jnp.concatenate(
            [jnp.sin(angle), jnp.cos(angle)], axis=-1
        )

    x = x_ref[...]
    e = x + pos_scr[...][None, :, :]
    u = jnp.mean(e, axis=-1, keepdims=True)
    d = e - u
    s = jnp.mean(d * d, axis=-1, keepdims=True)
    y = d * lax.rsqrt(s + EPS)
    o_ref[...] = y * w_ref[...][None, None, :] + b_ref[...][None, None, :]


def _tc_posln(x, ln_weight, ln_bias, S=8):
    return pl.pallas_call(
        _tc_posln_body,
        grid=(B // S,),
        in_specs=[
            pl.BlockSpec((S, L, D), lambda i: (i, 0, 0)),
            pl.BlockSpec((D,), lambda i: (0,)),
            pl.BlockSpec((D,), lambda i: (0,)),
        ],
        out_specs=pl.BlockSpec((S, L, D), lambda i: (i, 0, 0)),
        out_shape=jax.ShapeDtypeStruct((B, L, D), jnp.float32),
        scratch_shapes=[pltpu.VMEM((L, D), jnp.float32)],
    )(x, ln_weight, ln_bias)


def kernel(input_ids, table, ln_weight, ln_bias):
    ids = input_ids.astype(jnp.int32).reshape(NW, N_CHUNKS, CH)
    rows = _make_sc_gather()(ids, table)
    return _tc_posln(rows.reshape(B, L, D), ln_weight, ln_bias)


# speedup vs baseline: 1.1657x; 1.1657x over previous
import functools

import jax
import jax.numpy as jnp
from jax import lax
from jax.experimental import pallas as pl
from jax.experimental.pallas import tpu as pltpu
from jax.experimental.pallas import tpu_sc as plsc

VOCAB = 100000
D = 128
B = 1024
L = 200
EPS = 1e-12

NC = 2
NS = 16
NW = NC * NS
N_TOK = B * L
TOK_PER_W = N_TOK // NW
CH = 128
N_CHUNKS = TOK_PER_W // CH
NBUF = 2

@functools.cache
def _make_sc_gather():
    mesh = plsc.VectorSubcoreMesh(
        core_axis_name="c", subcore_axis_name="s", num_cores=NC, num_subcores=NS
    )
    return functools.partial(
        pl.kernel,
        out_type=jax.ShapeDtypeStruct((N_TOK, D), jnp.float32),
        mesh=mesh,
        scratch_types=[
            pltpu.VMEM((N_CHUNKS, CH), jnp.int32),
            pltpu.VMEM((NBUF, CH, D), jnp.float32),
            pltpu.SemaphoreType.DMA,
            pltpu.SemaphoreType.DMA,
        ],
    )(_sc_gather_body)


def _sc_gather_body(ids_hbm, table_hbm, out_hbm, idx_v, rows_v, sem0, sem1):
    wid = lax.axis_index("s") * NC + lax.axis_index("c")
    out_base = wid * TOK_PER_W
    sems = (sem0, sem1)
    pltpu.sync_copy(ids_hbm.at[wid], idx_v)

    def start(chunk, buf):
        return pltpu.async_copy(
            table_hbm.at[idx_v.at[chunk]], rows_v.at[buf], sems[buf]
        )

    for b in range(NBUF):
        start(b, b)

    def body(c, carry):
        for b in range(NBUF):
            chunk = c + b
            pltpu.make_async_copy(
                table_hbm.at[idx_v.at[chunk]], rows_v.at[b], sems[b]
            ).wait()
            pltpu.sync_copy(
                rows_v.at[b], out_hbm.at[pl.ds(out_base + chunk * CH, CH)]
            )

            @pl.when(chunk + NBUF < N_CHUNKS)
            def _():
                start(chunk + NBUF, b)

        return carry

    lax.fori_loop(0, N_CHUNKS // NBUF, lambda i, cy: body(i * NBUF, cy), 0,
                  unroll=False)


def _tc_posln_body(x_ref, w_ref, b_ref, o_ref, pos_scr):
    @pl.when(pl.program_id(0) == 0)
    def _():
        l_idx = lax.broadcasted_iota(jnp.int32, (L, D // 2), 0).astype(jnp.float32)
        j_idx = lax.broadcasted_iota(jnp.int32, (L, D // 2), 1).astype(jnp.float32)
        inv_freq = jnp.exp(j_idx * (-2.0 / D * jnp.log(10000.0)))
        angle = (L - 1.0 - l_idx) * inv_freq
        pos_scr[...] = jnp.concatenate(
            [jnp.sin(angle), jnp.cos(angle)], axis=-1
        )

    x = x_ref[...]
    e = x + pos_scr[...][None, :, :]
    u = jnp.mean(e, axis=-1, keepdims=True)
    d = e - u
    s = jnp.mean(d * d, axis=-1, keepdims=True)
    y = d * lax.rsqrt(s + EPS)
    o_ref[...] = y * w_ref[...][None, None, :] + b_ref[...][None, None, :]


def _tc_posln(x, ln_weight, ln_bias, S=16):
    return pl.pallas_call(
        _tc_posln_body,
        grid=(B // S,),
        in_specs=[
            pl.BlockSpec((S, L, D), lambda i: (i, 0, 0)),
            pl.BlockSpec((D,), lambda i: (0,)),
            pl.BlockSpec((D,), lambda i: (0,)),
        ],
        out_specs=pl.BlockSpec((S, L, D), lambda i: (i, 0, 0)),
        out_shape=jax.ShapeDtypeStruct((B, L, D), jnp.float32),
        scratch_shapes=[pltpu.VMEM((L, D), jnp.float32)],
    )(x, ln_weight, ln_bias)


def kernel(input_ids, table, ln_weight, ln_bias):
    ids = input_ids.astype(jnp.int32).reshape(NW, N_CHUNKS, CH)
    rows = _make_sc_gather()(ids, table)
    return _tc_posln(rows.reshape(B, L, D), ln_weight, ln_bias)


# speedup vs baseline: 1.2805x; 1.0985x over previous
import functools

import jax
import jax.numpy as jnp
from jax import lax
from jax.experimental import pallas as pl
from jax.experimental.pallas import tpu as pltpu
from jax.experimental.pallas import tpu_sc as plsc

VOCAB = 100000
D = 128
B = 1024
L = 200
EPS = 1e-12

NC = 2
NS = 16
NW = NC * NS
N_TOK = B * L
TOK_PER_W = N_TOK // NW
CH = 128
N_CHUNKS = TOK_PER_W // CH
NBUF = 2

@functools.cache
def _make_sc_gather():
    mesh = plsc.VectorSubcoreMesh(
        core_axis_name="c", subcore_axis_name="s", num_cores=NC, num_subcores=NS
    )
    return functools.partial(
        pl.kernel,
        out_type=jax.ShapeDtypeStruct((N_TOK, D), jnp.float32),
        mesh=mesh,
        scratch_types=[
            pltpu.VMEM((N_CHUNKS, CH), jnp.int32),
            pltpu.VMEM((NBUF, CH, D), jnp.float32),
            pltpu.SemaphoreType.DMA,
            pltpu.SemaphoreType.DMA,
        ],
    )(_sc_gather_body)


def _sc_gather_body(ids_hbm, table_hbm, out_hbm, idx_v, rows_v, sem0, sem1):
    wid = lax.axis_index("s") * NC + lax.axis_index("c")
    out_base = wid * TOK_PER_W
    sems = (sem0, sem1)
    pltpu.sync_copy(ids_hbm.at[wid], idx_v)

    def start(chunk, buf):
        return pltpu.async_copy(
            table_hbm.at[idx_v.at[chunk]], rows_v.at[buf], sems[buf]
        )

    for b in range(NBUF):
        start(b, b)

    def body(c, carry):
        for b in range(NBUF):
            chunk = c + b
            pltpu.make_async_copy(
                table_hbm.at[idx_v.at[chunk]], rows_v.at[b], sems[b]
            ).wait()
            pltpu.sync_copy(
                rows_v.at[b], out_hbm.at[pl.ds(out_base + chunk * CH, CH)]
            )

            @pl.when(chunk + NBUF < N_CHUNKS)
            def _():
                start(chunk + NBUF, b)

        return carry

    lax.fori_loop(0, N_CHUNKS // NBUF, lambda i, cy: body(i * NBUF, cy), 0,
                  unroll=False)


def _tc_posln_body(x_ref, w_ref, b_ref, o_ref, pos_scr):
    @pl.when(pl.program_id(0) == 0)
    def _():
        l_idx = lax.broadcasted_iota(jnp.int32, (L, D // 2), 0).astype(jnp.float32)
        j_idx = lax.broadcasted_iota(jnp.int32, (L, D // 2), 1).astype(jnp.float32)
        inv_freq = jnp.exp(j_idx * (-2.0 / D * jnp.log(10000.0)))
        angle = (L - 1.0 - l_idx) * inv_freq
        pos_scr[...] = jnp.concatenate(
            [jnp.sin(angle), jnp.cos(angle)], axis=-1
        )

    x = x_ref[...]
    e = x + pos_scr[...][None, :, :]
    u = jnp.mean(e, axis=-1, keepdims=True)
    d = e - u
    s = jnp.mean(d * d, axis=-1, keepdims=True)
    y = d * lax.rsqrt(s + EPS)
    o_ref[...] = y * w_ref[...][None, None, :] + b_ref[...][None, None, :]


def _tc_posln(x, ln_weight, ln_bias, S=32):
    return pl.pallas_call(
        _tc_posln_body,
        grid=(B // S,),
        in_specs=[
            pl.BlockSpec((S, L, D), lambda i: (i, 0, 0)),
            pl.BlockSpec((D,), lambda i: (0,)),
            pl.BlockSpec((D,), lambda i: (0,)),
        ],
        out_specs=pl.BlockSpec((S, L, D), lambda i: (i, 0, 0)),
        out_shape=jax.ShapeDtypeStruct((B, L, D), jnp.float32),
        scratch_shapes=[pltpu.VMEM((L, D), jnp.float32)],
    )(x, ln_weight, ln_bias)


def kernel(input_ids, table, ln_weight, ln_bias):
    ids = input_ids.astype(jnp.int32).reshape(NW, N_CHUNKS, CH)
    rows = _make_sc_gather()(ids, table)
    return _tc_posln(rows.reshape(B, L, D), ln_weight, ln_bias)


# speedup vs baseline: 1.3422x; 1.0482x over previous
import functools

import jax
import jax.numpy as jnp
from jax import lax
from jax.experimental import pallas as pl
from jax.experimental.pallas import tpu as pltpu
from jax.experimental.pallas import tpu_sc as plsc

VOCAB = 100000
D = 128
B = 1024
L = 200
EPS = 1e-12

NC = 2
NS = 16
NW = NC * NS
N_TOK = B * L
TOK_PER_W = N_TOK // NW
CH = 128
N_CHUNKS = TOK_PER_W // CH
NBUF = 2

@functools.cache
def _make_sc_gather():
    mesh = plsc.VectorSubcoreMesh(
        core_axis_name="c", subcore_axis_name="s", num_cores=NC, num_subcores=NS
    )
    return functools.partial(
        pl.kernel,
        out_type=jax.ShapeDtypeStruct((N_TOK, D), jnp.float32),
        mesh=mesh,
        scratch_types=[
            pltpu.VMEM((N_CHUNKS, CH), jnp.int32),
            pltpu.VMEM((NBUF, CH, D), jnp.float32),
            pltpu.SemaphoreType.DMA,
            pltpu.SemaphoreType.DMA,
        ],
    )(_sc_gather_body)


def _sc_gather_body(ids_hbm, table_hbm, out_hbm, idx_v, rows_v, sem0, sem1):
    wid = lax.axis_index("s") * NC + lax.axis_index("c")
    out_base = wid * TOK_PER_W
    sems = (sem0, sem1)
    pltpu.sync_copy(ids_hbm.at[wid], idx_v)

    def start(chunk, buf):
        return pltpu.async_copy(
            table_hbm.at[idx_v.at[chunk]], rows_v.at[buf], sems[buf]
        )

    for b in range(NBUF):
        start(b, b)

    def body(c, carry):
        for b in range(NBUF):
            chunk = c + b
            pltpu.make_async_copy(
                table_hbm.at[idx_v.at[chunk]], rows_v.at[b], sems[b]
            ).wait()
            pltpu.sync_copy(
                rows_v.at[b], out_hbm.at[pl.ds(out_base + chunk * CH, CH)]
            )

            @pl.when(chunk + NBUF < N_CHUNKS)
            def _():
                start(chunk + NBUF, b)

        return carry

    lax.fori_loop(0, N_CHUNKS // NBUF, lambda i, cy: body(i * NBUF, cy), 0,
                  unroll=False)


def _tc_posln_body(x_ref, w_ref, b_ref, o_ref, pos_scr):
    @pl.when(pl.program_id(0) == 0)
    def _():
        l_idx = lax.broadcasted_iota(jnp.int32, (L, D // 2), 0).astype(jnp.float32)
        j_idx = lax.broadcasted_iota(jnp.int32, (L, D // 2), 1).astype(jnp.float32)
        inv_freq = jnp.exp(j_idx * (-2.0 / D * jnp.log(10000.0)))
        angle = (L - 1.0 - l_idx) * inv_freq
        pos_scr[...] = jnp.concatenate(
            [jnp.sin(angle), jnp.cos(angle)], axis=-1
        )

    x = x_ref[...]
    e = x + pos_scr[...][None, :, :]
    u = jnp.mean(e, axis=-1, keepdims=True)
    d = e - u
    s = jnp.mean(d * d, axis=-1, keepdims=True)
    y = d * lax.rsqrt(s + EPS)
    o_ref[...] = y * w_ref[...][None, None, :] + b_ref[...][None, None, :]


def _tc_posln(x, ln_weight, ln_bias, S=64):
    return pl.pallas_call(
        _tc_posln_body,
        grid=(B // S,),
        in_specs=[
            pl.BlockSpec((S, L, D), lambda i: (i, 0, 0)),
            pl.BlockSpec((D,), lambda i: (0,)),
            pl.BlockSpec((D,), lambda i: (0,)),
        ],
        out_specs=pl.BlockSpec((S, L, D), lambda i: (i, 0, 0)),
        out_shape=jax.ShapeDtypeStruct((B, L, D), jnp.float32),
        scratch_shapes=[pltpu.VMEM((L, D), jnp.float32)],
    )(x, ln_weight, ln_bias)


def kernel(input_ids, table, ln_weight, ln_bias):
    ids = input_ids.astype(jnp.int32).reshape(NW, N_CHUNKS, CH)
    rows = _make_sc_gather()(ids, table)
    return _tc_posln(rows.reshape(B, L, D), ln_weight, ln_bias)


# speedup vs baseline: 1.3933x; 1.0380x over previous
import functools

import jax
import jax.numpy as jnp
from jax import lax
from jax.experimental import pallas as pl
from jax.experimental.pallas import tpu as pltpu
from jax.experimental.pallas import tpu_sc as plsc

VOCAB = 100000
D = 128
B = 1024
L = 200
EPS = 1e-12

NC = 2
NS = 16
NW = NC * NS
N_TOK = B * L
NBUF = 2

K_PIPE = 2
TOK_PER_CALL = N_TOK // K_PIPE
TOK_PER_W = TOK_PER_CALL // NW
CH = 128
N_CHUNKS = TOK_PER_W // CH
assert TOK_PER_W % CH == 0 and TOK_PER_CALL % L == 0


@functools.cache
def _make_sc_gather():
    mesh = plsc.VectorSubcoreMesh(
        core_axis_name="c", subcore_axis_name="s", num_cores=NC, num_subcores=NS
    )
    return functools.partial(
        pl.kernel,
        out_type=jax.ShapeDtypeStruct((TOK_PER_CALL, D), jnp.float32),
        mesh=mesh,
        scratch_types=[
            pltpu.VMEM((N_CHUNKS, CH), jnp.int32),
            pltpu.VMEM((NBUF, CH, D), jnp.float32),
            pltpu.SemaphoreType.DMA,
            pltpu.SemaphoreType.DMA,
        ],
    )(_sc_gather_body)


def _sc_gather_body(ids_hbm, table_hbm, out_hbm, idx_v, rows_v, sem0, sem1):
    wid = lax.axis_index("s") * NC + lax.axis_index("c")
    out_base = wid * TOK_PER_W
    sems = (sem0, sem1)
    pltpu.sync_copy(ids_hbm.at[wid], idx_v)

    def start(chunk, buf):
        return pltpu.async_copy(
            table_hbm.at[idx_v.at[chunk]], rows_v.at[buf], sems[buf]
        )

    for b in range(NBUF):
        start(b, b)

    def body(c, carry):
        for b in range(NBUF):
            chunk = c + b
            pltpu.make_async_copy(
                table_hbm.at[idx_v.at[chunk]], rows_v.at[b], sems[b]
            ).wait()
            pltpu.sync_copy(
                rows_v.at[b], out_hbm.at[pl.ds(out_base + chunk * CH, CH)]
            )

            @pl.when(chunk + NBUF < N_CHUNKS)
            def _():
                start(chunk + NBUF, b)

        return carry

    lax.fori_loop(0, N_CHUNKS // NBUF, lambda i, cy: body(i * NBUF, cy), 0,
                  unroll=False)


def _tc_posln_body_first(x_ref, w_ref, b_ref, o_ref, pos_scr):
    _tc_posln_body(x_ref, w_ref, b_ref, o_ref, pos_scr)


def _tc_posln_body_chained(prev_ref, x_ref, w_ref, b_ref, o_ref, pos_scr):
    del prev_ref
    _tc_posln_body(x_ref, w_ref, b_ref, o_ref, pos_scr)


def _tc_posln_body(x_ref, w_ref, b_ref, o_ref, pos_scr):
    @pl.when(pl.program_id(0) == 0)
    def _():
        l_idx = lax.broadcasted_iota(jnp.int32, (L, D // 2), 0).astype(jnp.float32)
        j_idx = lax.broadcasted_iota(jnp.int32, (L, D // 2), 1).astype(jnp.float32)
        inv_freq = jnp.exp(j_idx * (-2.0 / D * jnp.log(10000.0)))
        angle = (L - 1.0 - l_idx) * inv_freq
        pos_scr[...] = jnp.concatenate(
            [jnp.sin(angle), jnp.cos(angle)], axis=-1
        )

    x = x_ref[...]
    e = x + pos_scr[...][None, :, :]
    u = jnp.mean(e, axis=-1, keepdims=True)
    d = e - u
    s = jnp.mean(d * d, axis=-1, keepdims=True)
    y = d * lax.rsqrt(s + EPS)
    o_ref[...] = y * w_ref[...][None, None, :] + b_ref[...][None, None, :]


def _tc_posln_chunk(x, prev, c, ln_weight, ln_bias, S=64):
    nblk = (B // K_PIPE) // S
    base = c * nblk
    x_spec = pl.BlockSpec((S, L, D), lambda i: (i, 0, 0))
    wb_spec = pl.BlockSpec((D,), lambda i: (0,))
    out_spec = pl.BlockSpec((S, L, D), lambda i, _b=base: (i + _b, 0, 0))
    common = dict(
        grid=(nblk,),
        out_specs=out_spec,
        out_shape=jax.ShapeDtypeStruct((B, L, D), jnp.float32),
        scratch_shapes=[pltpu.VMEM((L, D), jnp.float32)],
    )
    if prev is None:
        return pl.pallas_call(
            _tc_posln_body_first,
            in_specs=[x_spec, wb_spec, wb_spec],
            **common,
        )(x, ln_weight, ln_bias)
    return pl.pallas_call(
        _tc_posln_body_chained,
        in_specs=[pl.BlockSpec(memory_space=pl.ANY), x_spec, wb_spec, wb_spec],
        input_output_aliases={0: 0},
        **common,
    )(prev, x, ln_weight, ln_bias)


def kernel(input_ids, table, ln_weight, ln_bias):
    ids = input_ids.astype(jnp.int32).reshape(K_PIPE, NW, N_CHUNKS, CH)
    gather = _make_sc_gather()
    out = None
    for c in range(K_PIPE):
        rows = gather(ids[c], table)
        out = _tc_posln_chunk(
            rows.reshape(B // K_PIPE, L, D), out, c, ln_weight, ln_bias
        )
    return out
